# Initial kernel scaffold; baseline (speedup 1.0000x reference)
import jax
import jax.numpy as jnp
from jax.experimental import pallas as pl


def kernel(node_feature_mat, edge_feature_mat, adj_max, Wm_p, bm_p, Wu_p, bu_p, Wo_p, bo_p, Wm_v, bm_v, Wu_v, bu_v, Wo_v, bo_v):
    raise NotImplementedError("write your pallas kernel here")


# speedup vs baseline: 2.9380x; 2.9380x over previous
import jax
import jax.numpy as jnp
from jax.experimental import pallas as pl
from jax.experimental.pallas import tpu as pltpu

B, N, F, E, A, H, DIAMETER = 32, 64, 128, 16, 32, 128, 3
BG = 2

_BIG = 1e30


def _mpnn_branch(h0, e2, adj, We, bm, Wi, Wj, Wuh, Wua, bu):
    eW = jnp.dot(e2, We, preferred_element_type=jnp.float32)
    eWm = eW.reshape(BG, N, N, H) + bm + ((adj - 1.0) * _BIG)[..., None]
    h = h0
    for _ in range(DIAMETER):
        ai = jnp.dot(h, Wi, preferred_element_type=jnp.float32)
        aj = jnp.dot(h, Wj, preferred_element_type=jnp.float32)
        m = jax.nn.relu(ai.reshape(BG, N, 1, H) + aj.reshape(BG, 1, N, H) + eWm)
        agg = jnp.sum(m, axis=2).reshape(BG * N, H)
        h = jax.nn.relu(
            jnp.dot(h, Wuh, preferred_element_type=jnp.float32)
            + jnp.dot(agg, Wua, preferred_element_type=jnp.float32)
            + bu
        )
    return jnp.sum(h.reshape(BG, N, F), axis=1)


def _kernel(node_ref, e_ref, adj_ref,
            Wi_p, Wj_p, We_p, bm_p, Wuh_p, Wua_p, bu_p, Wo_p, bo_p,
            Wi_v, Wj_v, We_v, bm_v, Wuh_v, Wua_v, bu_v, WovT, bo_v,
            out_p, out_v):
    h0 = node_ref[...]
    e2 = e_ref[...]
    adj = adj_ref[...]

    pooled_p = _mpnn_branch(h0, e2, adj, We_p[...], bm_p[...], Wi_p[...],
                            Wj_p[...], Wuh_p[...], Wua_p[...], bu_p[...])
    out_p[...] = (jnp.dot(pooled_p, Wo_p[...],
                          preferred_element_type=jnp.float32)
                  + bo_p[...]).reshape(1, BG, A)

    pooled_v = _mpnn_branch(h0, e2, adj, We_v[...], bm_v[...], Wi_v[...],
                            Wj_v[...], Wuh_v[...], Wua_v[...], bu_v[...])
    out_v[...] = (jnp.sum(pooled_v * WovT[...], axis=1, keepdims=True)
                  + bo_v[...]).reshape(1, BG, 1)


@jax.jit
def kernel(node_feature_mat, edge_feature_mat, adj_max,
           Wm_p, bm_p, Wu_p, bu_p, Wo_p, bo_p,
           Wm_v, bm_v, Wu_v, bu_v, Wo_v, bo_v):
    node2 = node_feature_mat.reshape(B * N, F)
    e2 = edge_feature_mat.reshape(B * N * N, E)
    adj_f = adj_max.astype(jnp.float32)

    def wsplit(Wm, Wu):
        return (Wm[:F], Wm[F:2 * F], Wm[2 * F:], Wu[:F], Wu[F:])

    Wi_p, Wj_p, We_p, Wuh_p, Wua_p = wsplit(Wm_p, Wu_p)
    Wi_v, Wj_v, We_v, Wuh_v, Wua_v = wsplit(Wm_v, Wu_v)

    rep = lambda *s: pl.BlockSpec(s, lambda i: (0,) * len(s))
    grid = B // BG

    out_p, out_v = pl.pallas_call(
        _kernel,
        grid=(grid,),
        in_specs=[
            pl.BlockSpec((BG * N, F), lambda i: (i, 0)),
            pl.BlockSpec((BG * N * N, E), lambda i: (i, 0)),
            pl.BlockSpec((BG, N, N), lambda i: (i, 0, 0)),
            rep(F, H), rep(F, H), rep(E, H), rep(1, H),
            rep(F, F), rep(H, F), rep(1, F), rep(F, A), rep(1, A),
            rep(F, H), rep(F, H), rep(E, H), rep(1, H),
            rep(F, F), rep(H, F), rep(1, F), rep(1, F), rep(1, 1),
        ],
        out_specs=[
            pl.BlockSpec((1, BG, A), lambda i: (i, 0, 0)),
            pl.BlockSpec((1, BG, 1), lambda i: (i, 0, 0)),
        ],
        out_shape=[
            jax.ShapeDtypeStruct((B // BG, BG, A), jnp.float32),
            jax.ShapeDtypeStruct((B // BG, BG, 1), jnp.float32),
        ],
        compiler_params=pltpu.CompilerParams(
            dimension_semantics=("parallel",),
        ),
    )(node2, e2, adj_f,
      Wi_p, Wj_p, We_p, bm_p.reshape(1, H), Wuh_p, Wua_p,
      bu_p.reshape(1, F), Wo_p, bo_p.reshape(1, A),
      Wi_v, Wj_v, We_v, bm_v.reshape(1, H), Wuh_v, Wua_v,
      bu_v.reshape(1, F), Wo_v.reshape(1, F), bo_v.reshape(1, 1))

    return out_p.reshape(B, A), out_v.reshape(-1)


# speedup vs baseline: 3.1671x; 1.0780x over previous
import jax
import jax.numpy as jnp
from jax.experimental import pallas as pl
from jax.experimental.pallas import tpu as pltpu

B, N, F, E, A, H, DIAMETER = 32, 64, 128, 16, 32, 128, 3
BG = 2
JC = 8

_BIG = 1e30


def _mpnn_branch(h0, e2, adj, We, bm, Wi, Wj, Wuh, Wua, bu):
    eW = jnp.dot(e2, We, preferred_element_type=jnp.float32)
    eWm = eW.reshape(BG, N, N, H) + ((adj - 1.0) * _BIG)[..., None]
    h = h0
    for _ in range(DIAMETER):
        ai = jnp.dot(h, Wi, preferred_element_type=jnp.float32) + bm
        aj = jnp.dot(h, Wj, preferred_element_type=jnp.float32)
        ai4 = ai.reshape(BG, N, 1, H)
        acc = jnp.zeros((BG, N, JC, H), dtype=jnp.float32)
        for jc in range(0, N, JC):
            aj_c = aj.reshape(BG, N, H)[:, jc:jc + JC, :].reshape(BG, 1, JC, H)
            acc = acc + jax.nn.relu(ai4 + aj_c + eWm[:, :, jc:jc + JC, :])
        agg = jnp.sum(acc, axis=2)
        h = jax.nn.relu(
            jnp.dot(h, Wuh, preferred_element_type=jnp.float32)
            + jnp.dot(agg.reshape(BG * N, H), Wua,
                      preferred_element_type=jnp.float32)
            + bu
        )
    return jnp.sum(h.reshape(BG, N, F), axis=1)


def _kernel(node_ref, e_ref, adj_ref,
            Wi_p, Wj_p, We_p, bm_p, Wuh_p, Wua_p, bu_p, Wo_p, bo_p,
            Wi_v, Wj_v, We_v, bm_v, Wuh_v, Wua_v, bu_v, WovT, bo_v,
            out_p, out_v):
    h0 = node_ref[...]
    e2 = e_ref[...]
    adj = adj_ref[...]

    pooled_p = _mpnn_branch(h0, e2, adj, We_p[...], bm_p[...], Wi_p[...],
                            Wj_p[...], Wuh_p[...], Wua_p[...], bu_p[...])
    out_p[...] = (jnp.dot(pooled_p, Wo_p[...],
                          preferred_element_type=jnp.float32)
                  + bo_p[...]).reshape(1, BG, A)

    pooled_v = _mpnn_branch(h0, e2, adj, We_v[...], bm_v[...], Wi_v[...],
                            Wj_v[...], Wuh_v[...], Wua_v[...], bu_v[...])
    out_v[...] = (jnp.sum(pooled_v * WovT[...], axis=1, keepdims=True)
                  + bo_v[...]).reshape(1, BG, 1)


@jax.jit
def kernel(node_feature_mat, edge_feature_mat, adj_max,
           Wm_p, bm_p, Wu_p, bu_p, Wo_p, bo_p,
           Wm_v, bm_v, Wu_v, bu_v, Wo_v, bo_v):
    node2 = node_feature_mat.reshape(B * N, F)
    e2 = edge_feature_mat.reshape(B * N * N, E)
    adj_f = adj_max.astype(jnp.float32)

    def wsplit(Wm, Wu):
        return (Wm[:F], Wm[F:2 * F], Wm[2 * F:], Wu[:F], Wu[F:])

    Wi_p, Wj_p, We_p, Wuh_p, Wua_p = wsplit(Wm_p, Wu_p)
    Wi_v, Wj_v, We_v, Wuh_v, Wua_v = wsplit(Wm_v, Wu_v)

    rep = lambda *s: pl.BlockSpec(s, lambda i: (0,) * len(s))
    grid = B // BG

    out_p, out_v = pl.pallas_call(
        _kernel,
        grid=(grid,),
        in_specs=[
            pl.BlockSpec((BG * N, F), lambda i: (i, 0)),
            pl.BlockSpec((BG * N * N, E), lambda i: (i, 0)),
            pl.BlockSpec((BG, N, N), lambda i: (i, 0, 0)),
            rep(F, H), rep(F, H), rep(E, H), rep(1, H),
            rep(F, F), rep(H, F), rep(1, F), rep(F, A), rep(1, A),
            rep(F, H), rep(F, H), rep(E, H), rep(1, H),
            rep(F, F), rep(H, F), rep(1, F), rep(1, F), rep(1, 1),
        ],
        out_specs=[
            pl.BlockSpec((1, BG, A), lambda i: (i, 0, 0)),
            pl.BlockSpec((1, BG, 1), lambda i: (i, 0, 0)),
        ],
        out_shape=[
            jax.ShapeDtypeStruct((B // BG, BG, A), jnp.float32),
            jax.ShapeDtypeStruct((B // BG, BG, 1), jnp.float32),
        ],
        compiler_params=pltpu.CompilerParams(
            dimension_semantics=("parallel",),
        ),
    )(node2, e2, adj_f,
      Wi_p, Wj_p, We_p, bm_p.reshape(1, H), Wuh_p, Wua_p,
      bu_p.reshape(1, F), Wo_p, bo_p.reshape(1, A),
      Wi_v, Wj_v, We_v, bm_v.reshape(1, H), Wuh_v, Wua_v,
      bu_v.reshape(1, F), Wo_v.reshape(1, F), bo_v.reshape(1, 1))

    return out_p.reshape(B, A), out_v.reshape(-1)
